# Initial kernel scaffold; baseline (speedup 1.0000x reference)
import jax
import jax.numpy as jnp
from jax.experimental import pallas as pl


def kernel(x, edge_index, W_key, b_key, W_query, b_query, W_value, b_value, W_skip, bias):
    raise NotImplementedError("write your pallas kernel here")


# speedup vs baseline: 1.5470x; 1.5470x over previous
import functools

import jax
import jax.numpy as jnp
from jax import lax
from jax.experimental import pallas as pl
from jax.experimental.pallas import tpu as pltpu
from jax.experimental.pallas import tpu_sc as plsc

N = 10000
E = 320000
D = 128

NPAD = 10240
ROWS_PER_TILE = NPAD // 16
NWORKERS = 32
EPW = E // NWORKERS
C = 80
NCHUNKS = EPW // C
BLK = 256


def _proj_body(x_ref, wk, bk, wq, bq, wv, bv, ws, bb, kd_ref, qv_ref, skip_ref):
    x = x_ref[...]
    kd_ref[...] = jnp.dot(x, wk[...], preferred_element_type=jnp.float32) + bk[...]
    qv_ref[:, : D] = jnp.dot(x, wq[...], preferred_element_type=jnp.float32) + bq[...]
    qv_ref[:, D:] = jnp.dot(x, wv[...], preferred_element_type=jnp.float32) + bv[...]
    skip_ref[...] = jnp.dot(x, ws[...], preferred_element_type=jnp.float32) + bb[...]


def _proj(x_pad, Wk, bk, Wq, bq, Wv, bv, Ws, bb):
    grid = (NPAD // BLK,)
    w_spec = pl.BlockSpec((D, D), lambda i: (0, 0))
    b_spec = pl.BlockSpec((1, D), lambda i: (0, 0))
    return pl.pallas_call(
        _proj_body,
        grid=grid,
        in_specs=[
            pl.BlockSpec((BLK, D), lambda i: (i, 0)),
            w_spec, b_spec, w_spec, b_spec, w_spec, b_spec, w_spec, b_spec,
        ],
        out_specs=[
            pl.BlockSpec((BLK, D), lambda i: (i, 0)),
            pl.BlockSpec((BLK, 2 * D), lambda i: (i, 0)),
            pl.BlockSpec((BLK, D), lambda i: (i, 0)),
        ],
        out_shape=[
            jax.ShapeDtypeStruct((NPAD, D), jnp.float32),
            jax.ShapeDtypeStruct((NPAD, 2 * D), jnp.float32),
            jax.ShapeDtypeStruct((NPAD, D), jnp.float32),
        ],
    )(x_pad, Wk, bk, Wq, bq, Wv, bv, Ws, bb)


def _sc_body(kd, qv, skip, src, dst, out, dst_i, src_i, krow, qvrow, agg,
             sem1, sem2):
    cid = lax.axis_index("c")
    sid = lax.axis_index("s")
    wid = sid * 2 + cid
    rbase = sid * ROWS_PER_TILE

    @pl.when(cid == 0)
    def _():
        pltpu.sync_copy(skip.at[pl.ds(rbase, ROWS_PER_TILE)],
                        agg.at[pl.ds(rbase, ROWS_PER_TILE)])

    @pl.when(cid != 0)
    def _():
        zero = jnp.zeros((16,), jnp.float32)

        def zrow(e, carry):
            for j in range(D // 16):
                krow[e, pl.ds(j * 16, 16)] = zero
            return carry

        lax.fori_loop(0, C, zrow, 0)
        for r in range(ROWS_PER_TILE // C):
            pltpu.sync_copy(krow, agg.at[pl.ds(rbase + r * C, C)])

    plsc.subcore_barrier()

    ebase = wid * EPW

    def chunk(i, carry):
        off = ebase + i * C
        pltpu.sync_copy(dst.at[pl.ds(off, C)], dst_i)
        pltpu.sync_copy(src.at[pl.ds(off, C)], src_i)
        cp1 = pltpu.async_copy(kd.at[dst_i], krow, sem1)
        cp2 = pltpu.async_copy(qv.at[src_i], qvrow, sem2)
        cp1.wait()
        cp2.wait()

        def edge(e, c2):
            for j in range(D // 16):
                s = pl.ds(j * 16, 16)
                kx = krow[e, s]
                qx = qvrow[e, s]
                vx = qvrow[e, pl.ds(D + j * 16, 16)]
                eta = 1.0 / (1.0 + jnp.exp(-(kx + qx)))
                krow[e, s] = eta * vx
            return c2

        lax.fori_loop(0, C, edge, 0)
        pltpu.sync_copy(krow, agg.at[dst_i], add=True)
        return carry

    lax.fori_loop(0, NCHUNKS, chunk, 0)
    plsc.subcore_barrier()
    pltpu.sync_copy(agg.at[pl.ds(rbase, ROWS_PER_TILE)],
                    out.at[cid, pl.ds(rbase, ROWS_PER_TILE)])


@functools.partial(
    pl.kernel,
    mesh=plsc.VectorSubcoreMesh(core_axis_name="c", subcore_axis_name="s"),
    out_type=jax.ShapeDtypeStruct((2, NPAD, D), jnp.float32),
    scratch_types=[
        pltpu.VMEM((C,), jnp.int32),
        pltpu.VMEM((C,), jnp.int32),
        pltpu.VMEM((C, D), jnp.float32),
        pltpu.VMEM((C, 2 * D), jnp.float32),
        pltpu.VMEM_SHARED((NPAD, D), jnp.float32),
        pltpu.SemaphoreType.DMA,
        pltpu.SemaphoreType.DMA,
    ],
)
def _sc_msg(kd, qv, skip, src, dst, out, dst_i, src_i, krow, qvrow, agg,
            sem1, sem2):
    _sc_body(kd, qv, skip, src, dst, out, dst_i, src_i, krow, qvrow, agg,
             sem1, sem2)


def _add_body(a_ref, b_ref, o_ref):
    o_ref[...] = a_ref[...] + b_ref[...]


def _combine(p0, p1):
    grid = (NPAD // BLK,)
    spec = pl.BlockSpec((BLK, D), lambda i: (i, 0))
    return pl.pallas_call(
        _add_body,
        grid=grid,
        in_specs=[spec, spec],
        out_specs=spec,
        out_shape=jax.ShapeDtypeStruct((NPAD, D), jnp.float32),
    )(p0, p1)


def kernel(x, edge_index, W_key, b_key, W_query, b_query, W_value, b_value,
           W_skip, bias):
    x_pad = jnp.pad(x, ((0, NPAD - N), (0, 0)))
    kd, qv, skip = _proj(
        x_pad,
        W_key, b_key.reshape(1, D),
        W_query, b_query.reshape(1, D),
        W_value, b_value.reshape(1, D),
        W_skip, bias.reshape(1, D),
    )
    partials = _sc_msg(kd, qv, skip, edge_index[0], edge_index[1])
    out = _combine(partials[0], partials[1])
    return out[:N]


# speedup vs baseline: 1.7997x; 1.1633x over previous
import functools

import jax
import jax.numpy as jnp
from jax import lax
from jax.experimental import pallas as pl
from jax.experimental.pallas import tpu as pltpu
from jax.experimental.pallas import tpu_sc as plsc

N = 10000
E = 320000
D = 128

NPAD = 10240
ROWS_PER_TILE = NPAD // 16
NWORKERS = 32
EPW = E // NWORKERS
C = 40
NCHUNKS = EPW // C
SUPER = 25
NSUPER = NCHUNKS // SUPER
BLK = 256


def _proj_body(x_ref, wk, bk, wq, bq, wv, bv, ws, bb, kd_ref, qv_ref, skip_ref):
    x = x_ref[...]
    kd_ref[...] = jnp.dot(x, wk[...], preferred_element_type=jnp.float32) + bk[...]
    qv_ref[:, : D] = jnp.dot(x, wq[...], preferred_element_type=jnp.float32) + bq[...]
    qv_ref[:, D:] = jnp.dot(x, wv[...], preferred_element_type=jnp.float32) + bv[...]
    skip_ref[...] = jnp.dot(x, ws[...], preferred_element_type=jnp.float32) + bb[...]


def _proj(x_pad, Wk, bk, Wq, bq, Wv, bv, Ws, bb):
    grid = (NPAD // BLK,)
    w_spec = pl.BlockSpec((D, D), lambda i: (0, 0))
    b_spec = pl.BlockSpec((1, D), lambda i: (0, 0))
    return pl.pallas_call(
        _proj_body,
        grid=grid,
        in_specs=[
            pl.BlockSpec((BLK, D), lambda i: (i, 0)),
            w_spec, b_spec, w_spec, b_spec, w_spec, b_spec, w_spec, b_spec,
        ],
        out_specs=[
            pl.BlockSpec((BLK, D), lambda i: (i, 0)),
            pl.BlockSpec((BLK, 2 * D), lambda i: (i, 0)),
            pl.BlockSpec((BLK, D), lambda i: (i, 0)),
        ],
        out_shape=[
            jax.ShapeDtypeStruct((NPAD, D), jnp.float32),
            jax.ShapeDtypeStruct((NPAD, 2 * D), jnp.float32),
            jax.ShapeDtypeStruct((NPAD, D), jnp.float32),
        ],
    )(x_pad, Wk, bk, Wq, bq, Wv, bv, Ws, bb)


def _sc_body(kd, qv, skip, src4, dst4, out, dsti, srci,
             krs, qvs, agg, gsem, ssems):
    cid = lax.axis_index("c")
    sid = lax.axis_index("s")
    wid = sid * 2 + cid
    rbase = sid * ROWS_PER_TILE

    @pl.when(cid == 0)
    def _():
        pltpu.sync_copy(skip.at[pl.ds(rbase, ROWS_PER_TILE)],
                        agg.at[pl.ds(rbase, ROWS_PER_TILE)])

    @pl.when(cid != 0)
    def _():
        zero = jnp.zeros((16,), jnp.float32)

        def zrow(e, carry):
            for j in range(D // 16):
                krs[0][e, pl.ds(j * 16, 16)] = zero
            return carry

        lax.fori_loop(0, C, zrow, 0)
        for r in range(ROWS_PER_TILE // C):
            pltpu.sync_copy(krs[0], agg.at[pl.ds(rbase + r * C, C)])

    def load_super(k):
        pltpu.sync_copy(dst4.at[wid, k], dsti.at[k % 2])
        pltpu.sync_copy(src4.at[wid, k], srci.at[k % 2])

    def issue_gather(c, b):
        par, row = (c // SUPER) % 2, c % SUPER
        pltpu.async_copy(kd.at[dsti.at[par, row]], krs[b], gsem)
        pltpu.async_copy(qv.at[srci.at[par, row]], qvs[b], gsem)

    def wait_gather(b):
        pltpu.make_async_copy(kd.at[dsti.at[0, 0]], krs[b], gsem).wait()
        pltpu.make_async_copy(qv.at[srci.at[0, 0]], qvs[b], gsem).wait()

    def issue_scatter(c, b):
        par, row = (c // SUPER) % 2, c % SUPER
        pltpu.async_copy(krs[b], agg.at[dsti.at[par, row]], ssems[b],
                         add=True)

    def wait_scatter(b):
        pltpu.make_async_copy(krs[b], agg.at[dsti.at[0, 0]], ssems[b]).wait()

    def compute(b):
        kr, qvr = krs[b], qvs[b]

        def edge(e, c2):
            for j in range(D // 16):
                s = pl.ds(j * 16, 16)
                kx = kr[e, s]
                qx = qvr[e, s]
                vx = qvr[e, pl.ds(D + j * 16, 16)]
                eta = 1.0 / (1.0 + jnp.exp(-(kx + qx)))
                kr[e, s] = eta * vx
            return c2

        lax.fori_loop(0, C, edge, 0)

    load_super(0)
    issue_gather(0, 0)

    def step(c, s):
        nb = 1 - s

        @pl.when(jnp.logical_and((c + 1) % SUPER == 0, c + 1 < NCHUNKS))
        def _():
            load_super((c + 1) // SUPER)

        wait_gather(s)

        @pl.when(c >= 1)
        def _():
            wait_scatter(nb)

        @pl.when(c + 1 < NCHUNKS)
        def _():
            issue_gather(c + 1, nb)

        compute(s)
        issue_scatter(c, s)

    def body(i, carry):
        step(2 * i, 0)
        step(2 * i + 1, 1)
        return carry

    lax.fori_loop(0, (NCHUNKS - 2) // 2, body, 0)
    step(NCHUNKS - 2, 0)
    step(NCHUNKS - 1, 1)
    wait_scatter(1)

    plsc.subcore_barrier()
    pltpu.sync_copy(agg.at[pl.ds(rbase, ROWS_PER_TILE)],
                    out.at[cid, pl.ds(rbase, ROWS_PER_TILE)])


@functools.partial(
    pl.kernel,
    mesh=plsc.VectorSubcoreMesh(core_axis_name="c", subcore_axis_name="s"),
    out_type=jax.ShapeDtypeStruct((2, NPAD, D), jnp.float32),
    scratch_types=[
        pltpu.VMEM((2, SUPER, C), jnp.int32),
        pltpu.VMEM((2, SUPER, C), jnp.int32),
        pltpu.VMEM((C, D), jnp.float32),
        pltpu.VMEM((C, D), jnp.float32),
        pltpu.VMEM((C, 2 * D), jnp.float32),
        pltpu.VMEM((C, 2 * D), jnp.float32),
        pltpu.VMEM_SHARED((NPAD, D), jnp.float32),
        pltpu.SemaphoreType.DMA,
        pltpu.SemaphoreType.DMA,
        pltpu.SemaphoreType.DMA,
    ],
)
def _sc_msg(kd, qv, skip, src4, dst4, out, dsti, srci,
            kr0, kr1, qv0, qv1, agg, g0, s0, s1):
    _sc_body(kd, qv, skip, src4, dst4, out, dsti, srci,
             (kr0, kr1), (qv0, qv1), agg, g0, (s0, s1))


def _add_body(a_ref, b_ref, o_ref):
    o_ref[...] = a_ref[...] + b_ref[...]


def _combine(p0, p1):
    grid = (NPAD // BLK,)
    spec = pl.BlockSpec((BLK, D), lambda i: (i, 0))
    return pl.pallas_call(
        _add_body,
        grid=grid,
        in_specs=[spec, spec],
        out_specs=spec,
        out_shape=jax.ShapeDtypeStruct((NPAD, D), jnp.float32),
    )(p0, p1)


def kernel(x, edge_index, W_key, b_key, W_query, b_query, W_value, b_value,
           W_skip, bias):
    x_pad = jnp.pad(x, ((0, NPAD - N), (0, 0)))
    kd, qv, skip = _proj(
        x_pad,
        W_key, b_key.reshape(1, D),
        W_query, b_query.reshape(1, D),
        W_value, b_value.reshape(1, D),
        W_skip, bias.reshape(1, D),
    )
    src4 = edge_index[0].reshape(NWORKERS, NSUPER, SUPER, C)
    dst4 = edge_index[1].reshape(NWORKERS, NSUPER, SUPER, C)
    partials = _sc_msg(kd, qv, skip, src4, dst4)
    out = _combine(partials[0], partials[1])
    return out[:N]


# speedup vs baseline: 7.0713x; 3.9292x over previous
import functools

import jax
import jax.numpy as jnp
from jax import lax
from jax.experimental import pallas as pl
from jax.experimental.pallas import tpu as pltpu
from jax.experimental.pallas import tpu_sc as plsc

N = 10000
E = 320000
D = 128

NPAD = 10240
ROWS_PER_TILE = NPAD // 16
NWORKERS = 32
EPW = E // NWORKERS
C = 40
NCHUNKS = EPW // C
SUPER = 25
NSUPER = NCHUNKS // SUPER
BLK = 256


def _proj_body(x_ref, wk, bk, wq, bq, wv, bv, ws, bb, kd_ref, qv_ref, skip_ref):
    x = x_ref[...]
    kd_ref[...] = jnp.dot(x, wk[...], preferred_element_type=jnp.float32) + bk[...]
    qv_ref[:, : D] = jnp.dot(x, wq[...], preferred_element_type=jnp.float32) + bq[...]
    qv_ref[:, D:] = jnp.dot(x, wv[...], preferred_element_type=jnp.float32) + bv[...]
    skip_ref[...] = jnp.dot(x, ws[...], preferred_element_type=jnp.float32) + bb[...]


def _proj(x_pad, Wk, bk, Wq, bq, Wv, bv, Ws, bb):
    grid = (NPAD // BLK,)
    w_spec = pl.BlockSpec((D, D), lambda i: (0, 0))
    b_spec = pl.BlockSpec((1, D), lambda i: (0, 0))
    return pl.pallas_call(
        _proj_body,
        grid=grid,
        in_specs=[
            pl.BlockSpec((BLK, D), lambda i: (i, 0)),
            w_spec, b_spec, w_spec, b_spec, w_spec, b_spec, w_spec, b_spec,
        ],
        out_specs=[
            pl.BlockSpec((BLK, D), lambda i: (i, 0)),
            pl.BlockSpec((BLK, 2 * D), lambda i: (i, 0)),
            pl.BlockSpec((BLK, D), lambda i: (i, 0)),
        ],
        out_shape=[
            jax.ShapeDtypeStruct((NPAD, D), jnp.float32),
            jax.ShapeDtypeStruct((NPAD, 2 * D), jnp.float32),
            jax.ShapeDtypeStruct((NPAD, D), jnp.float32),
        ],
    )(x_pad, Wk, bk, Wq, bq, Wv, bv, Ws, bb)


def _sc_body(kd, qv, skip, src4, dst4, out, dsti, srci,
             krs, qvs, agg, gsem, ssems):
    cid = lax.axis_index("c")
    sid = lax.axis_index("s")
    wid = sid * 2 + cid
    rbase = sid * ROWS_PER_TILE

    @pl.when(cid == 0)
    def _():
        pltpu.sync_copy(skip.at[pl.ds(rbase, ROWS_PER_TILE)],
                        agg.at[pl.ds(rbase, ROWS_PER_TILE)])

    @pl.when(cid != 0)
    def _():
        zero = jnp.zeros((16,), jnp.float32)

        def zrow(e, carry):
            for j in range(D // 16):
                krs[0][e, pl.ds(j * 16, 16)] = zero
            return carry

        lax.fori_loop(0, C, zrow, 0)
        for r in range(ROWS_PER_TILE // C):
            pltpu.sync_copy(krs[0], agg.at[pl.ds(rbase + r * C, C)])

    def load_super(k):
        pltpu.sync_copy(dst4.at[wid, k], dsti.at[k % 2])
        pltpu.sync_copy(src4.at[wid, k], srci.at[k % 2])

    def issue_gather(c, b):
        par, row = (c // SUPER) % 2, c % SUPER
        pltpu.async_copy(kd.at[dsti.at[par, row]], krs[b], gsem)
        pltpu.async_copy(qv.at[srci.at[par, row]], qvs[b], gsem)

    def wait_gather(b):
        pltpu.make_async_copy(kd.at[dsti.at[0, 0]], krs[b], gsem).wait()
        pltpu.make_async_copy(qv.at[srci.at[0, 0]], qvs[b], gsem).wait()

    def issue_scatter(c, b):
        par, row = (c // SUPER) % 2, c % SUPER
        pltpu.async_copy(krs[b], agg.at[dsti.at[par, row]], ssems[b],
                         add=True)

    def wait_scatter(b):
        pltpu.make_async_copy(krs[b], agg.at[dsti.at[0, 0]], ssems[b]).wait()

    def compute(b):
        kr, qvr = krs[b], qvs[b]
        J = D // 16

        def edge(e, c2):
            kx = [kr[e, pl.ds(j * 16, 16)] for j in range(J)]
            qx = [qvr[e, pl.ds(j * 16, 16)] for j in range(J)]
            ex = [jnp.exp(-(kx[j] + qx[j])) for j in range(J)]
            vx = [qvr[e, pl.ds(D + j * 16, 16)] for j in range(J)]
            eta = [1.0 / (1.0 + ex[j]) for j in range(J)]
            for j in range(J):
                kr[e, pl.ds(j * 16, 16)] = eta[j] * vx[j]
            return c2

        lax.fori_loop(0, C, edge, 0, unroll=2)

    load_super(0)
    issue_gather(0, 0)

    def step(c, s):
        nb = 1 - s

        @pl.when(jnp.logical_and((c + 1) % SUPER == 0, c + 1 < NCHUNKS))
        def _():
            load_super((c + 1) // SUPER)

        wait_gather(s)

        @pl.when(c >= 1)
        def _():
            wait_scatter(nb)

        @pl.when(c + 1 < NCHUNKS)
        def _():
            issue_gather(c + 1, nb)

        compute(s)
        issue_scatter(c, s)

    def body(i, carry):
        step(2 * i, 0)
        step(2 * i + 1, 1)
        return carry

    lax.fori_loop(0, (NCHUNKS - 2) // 2, body, 0)
    step(NCHUNKS - 2, 0)
    step(NCHUNKS - 1, 1)
    wait_scatter(1)

    plsc.subcore_barrier()
    pltpu.sync_copy(agg.at[pl.ds(rbase, ROWS_PER_TILE)],
                    out.at[cid, pl.ds(rbase, ROWS_PER_TILE)])


@functools.partial(
    pl.kernel,
    mesh=plsc.VectorSubcoreMesh(core_axis_name="c", subcore_axis_name="s"),
    out_type=jax.ShapeDtypeStruct((2, NPAD, D), jnp.float32),
    scratch_types=[
        pltpu.VMEM((2, SUPER, C), jnp.int32),
        pltpu.VMEM((2, SUPER, C), jnp.int32),
        pltpu.VMEM((C, D), jnp.float32),
        pltpu.VMEM((C, D), jnp.float32),
        pltpu.VMEM((C, 2 * D), jnp.float32),
        pltpu.VMEM((C, 2 * D), jnp.float32),
        pltpu.VMEM_SHARED((NPAD, D), jnp.float32),
        pltpu.SemaphoreType.DMA,
        pltpu.SemaphoreType.DMA,
        pltpu.SemaphoreType.DMA,
    ],
)
def _sc_msg(kd, qv, skip, src4, dst4, out, dsti, srci,
            kr0, kr1, qv0, qv1, agg, g0, s0, s1):
    _sc_body(kd, qv, skip, src4, dst4, out, dsti, srci,
             (kr0, kr1), (qv0, qv1), agg, g0, (s0, s1))


def _add_body(a_ref, b_ref, o_ref):
    o_ref[...] = a_ref[...] + b_ref[...]


def _combine(p0, p1):
    grid = (NPAD // BLK,)
    spec = pl.BlockSpec((BLK, D), lambda i: (i, 0))
    return pl.pallas_call(
        _add_body,
        grid=grid,
        in_specs=[spec, spec],
        out_specs=spec,
        out_shape=jax.ShapeDtypeStruct((NPAD, D), jnp.float32),
    )(p0, p1)


def kernel(x, edge_index, W_key, b_key, W_query, b_query, W_value, b_value,
           W_skip, bias):
    x_pad = jnp.pad(x, ((0, NPAD - N), (0, 0)))
    kd, qv, skip = _proj(
        x_pad,
        W_key, b_key.reshape(1, D),
        W_query, b_query.reshape(1, D),
        W_value, b_value.reshape(1, D),
        W_skip, bias.reshape(1, D),
    )
    src4 = edge_index[0].reshape(NWORKERS, NSUPER, SUPER, C)
    dst4 = edge_index[1].reshape(NWORKERS, NSUPER, SUPER, C)
    partials = _sc_msg(kd, qv, skip, src4, dst4)
    out = _combine(partials[0], partials[1])
    return out[:N]
